# baseline (speedup 1.0000x reference)
import functools

import jax
import jax.numpy as jnp
from jax import lax
from jax.experimental import pallas as pl
from jax.experimental.pallas import tpu as pltpu
from jax.experimental.pallas import tpu_sc as plsc

_SC_CHUNK = 32
_SC_NCORES = 1
_SC_NBUF = 4
_TC_CHUNK = 1024
_TC_NBUF = 4


def _sc_part(flat, rows, feature_dim, tc_rows):
    info = plsc.get_sparse_core_info()
    nw = _SC_NCORES * info.num_subcores
    rpw = (rows - tc_rows) // nw
    ch = _SC_CHUNK
    nchunks = rpw // ch
    mesh = plsc.VectorSubcoreMesh(
        core_axis_name="c", subcore_axis_name="s", num_cores=_SC_NCORES)

    scratch = (
        [pltpu.VMEM((ch, feature_dim), jnp.float32) for _ in range(_SC_NBUF)]
        + [pltpu.SemaphoreType.DMA for _ in range(2 * _SC_NBUF)]
    )

    @functools.partial(
        pl.kernel,
        out_type=jax.ShapeDtypeStruct((rows, feature_dim), flat.dtype),
        mesh=mesh,
        scratch_types=scratch,
    )
    def sc_copy(in_hbm, out_hbm, *bufs_and_sems):
        bufs = bufs_and_sems[:_SC_NBUF]
        gsems = bufs_and_sems[_SC_NBUF:2 * _SC_NBUF]
        ssems = bufs_and_sems[2 * _SC_NBUF:]
        wid = lax.axis_index("s") * _SC_NCORES + lax.axis_index("c")
        base = tc_rows + wid * rpw

        def gather(i):
            b = i % _SC_NBUF
            return pltpu.make_async_copy(
                in_hbm.at[pl.ds(base + i * ch, ch)], bufs[b], gsems[b])

        def scatter(i):
            b = i % _SC_NBUF
            return pltpu.make_async_copy(
                bufs[b], out_hbm.at[pl.ds(base + i * ch, ch)], ssems[b])

        for i in range(_SC_NBUF - 1):
            gather(i).start()
        for i in range(nchunks):
            gather(i).wait()
            scatter(i).start()
            nxt = i + _SC_NBUF - 1
            if nxt < nchunks:
                if nxt - _SC_NBUF >= 0:
                    scatter(nxt - _SC_NBUF).wait()
                gather(nxt).start()
        for i in range(max(0, nchunks - _SC_NBUF), nchunks):
            scatter(i).wait()

    return sc_copy(flat)


def _tc_body(tc_rows, x_ref, prev_ref, o_ref, *bufs_and_sems):
    del prev_ref
    bufs = bufs_and_sems[:_TC_NBUF]
    gsems = bufs_and_sems[_TC_NBUF]
    ssems = bufs_and_sems[_TC_NBUF + 1]
    nchunks = tc_rows // _TC_CHUNK

    def gather(i):
        b = i % _TC_NBUF
        return pltpu.make_async_copy(
            x_ref.at[pl.ds(i * _TC_CHUNK, _TC_CHUNK)], bufs[b], gsems.at[b])

    def scatter(i):
        b = i % _TC_NBUF
        return pltpu.make_async_copy(
            bufs[b], o_ref.at[pl.ds(i * _TC_CHUNK, _TC_CHUNK)], ssems.at[b])

    for i in range(_TC_NBUF - 1):
        gather(i).start()
    for i in range(nchunks):
        gather(i).wait()
        scatter(i).start()
        nxt = i + _TC_NBUF - 1
        if nxt < nchunks:
            if nxt - _TC_NBUF >= 0:
                scatter(nxt - _TC_NBUF).wait()
            gather(nxt).start()
    for i in range(max(0, nchunks - _TC_NBUF), nchunks):
        scatter(i).wait()


def kernel(images_batch, masks_batch):
    del masks_batch
    B, ve_dim, feature_dim = images_batch.shape
    rows = B * ve_dim
    tc_rows = rows // 2
    flat = images_batch.reshape(rows, feature_dim)

    partial = _sc_part(flat, rows, feature_dim, tc_rows)

    out = pl.pallas_call(
        functools.partial(_tc_body, tc_rows),
        in_specs=[pl.BlockSpec(memory_space=pl.ANY),
                  pl.BlockSpec(memory_space=pl.ANY)],
        out_specs=pl.BlockSpec(memory_space=pl.ANY),
        scratch_shapes=(
            [pltpu.VMEM((_TC_CHUNK, feature_dim), jnp.float32)
             for _ in range(_TC_NBUF)]
            + [pltpu.SemaphoreType.DMA((_TC_NBUF,)),
               pltpu.SemaphoreType.DMA((_TC_NBUF,))]
        ),
        out_shape=jax.ShapeDtypeStruct((rows, feature_dim), flat.dtype),
        input_output_aliases={1: 0},
    )(flat, partial)
    return out.reshape(B, ve_dim, feature_dim)


# speedup vs baseline: 1.0865x; 1.0865x over previous
import functools

import jax
import jax.numpy as jnp
from jax import lax
from jax.experimental import pallas as pl
from jax.experimental.pallas import tpu as pltpu
from jax.experimental.pallas import tpu_sc as plsc

_SC_CHUNK = 32
_SC_NBUF = 4
_TC_CHUNK = 1024
_TC_NBUF = 4


def _sc_part(flat, rows, feature_dim, tc_rows):
    info = plsc.get_sparse_core_info()
    nw = info.num_cores * info.num_subcores
    rpw = (rows - tc_rows) // nw
    ch = _SC_CHUNK
    nchunks = rpw // ch
    mesh = plsc.VectorSubcoreMesh(core_axis_name="c", subcore_axis_name="s")

    scratch = (
        [pltpu.VMEM((ch, feature_dim), jnp.float32) for _ in range(_SC_NBUF)]
        + [pltpu.SemaphoreType.DMA for _ in range(2 * _SC_NBUF)]
    )

    @functools.partial(
        pl.kernel,
        out_type=jax.ShapeDtypeStruct((rows, feature_dim), flat.dtype),
        mesh=mesh,
        scratch_types=scratch,
    )
    def sc_copy(in_hbm, out_hbm, *bufs_and_sems):
        bufs = bufs_and_sems[:_SC_NBUF]
        gsems = bufs_and_sems[_SC_NBUF:2 * _SC_NBUF]
        ssems = bufs_and_sems[2 * _SC_NBUF:]
        wid = lax.axis_index("s") * info.num_cores + lax.axis_index("c")
        base = tc_rows + wid * rpw

        def gather(i):
            b = i % _SC_NBUF
            return pltpu.make_async_copy(
                in_hbm.at[pl.ds(base + i * ch, ch)], bufs[b], gsems[b])

        def scatter(i):
            b = i % _SC_NBUF
            return pltpu.make_async_copy(
                bufs[b], out_hbm.at[pl.ds(base + i * ch, ch)], ssems[b])

        for i in range(_SC_NBUF - 1):
            gather(i).start()
        for i in range(nchunks):
            gather(i).wait()
            scatter(i).start()
            nxt = i + _SC_NBUF - 1
            if nxt < nchunks:
                if nxt - _SC_NBUF >= 0:
                    scatter(nxt - _SC_NBUF).wait()
                gather(nxt).start()
        for i in range(max(0, nchunks - _SC_NBUF), nchunks):
            scatter(i).wait()

    return sc_copy(flat)


def _tc_body(tc_rows, x_ref, prev_ref, o_ref, *bufs_and_sems):
    del prev_ref
    bufs = bufs_and_sems[:_TC_NBUF]
    gsems = bufs_and_sems[_TC_NBUF]
    ssems = bufs_and_sems[_TC_NBUF + 1]
    nchunks = tc_rows // _TC_CHUNK

    def gather(i):
        b = i % _TC_NBUF
        return pltpu.make_async_copy(
            x_ref.at[pl.ds(i * _TC_CHUNK, _TC_CHUNK)], bufs[b], gsems.at[b])

    def scatter(i):
        b = i % _TC_NBUF
        return pltpu.make_async_copy(
            bufs[b], o_ref.at[pl.ds(i * _TC_CHUNK, _TC_CHUNK)], ssems.at[b])

    for i in range(_TC_NBUF - 1):
        gather(i).start()
    for i in range(nchunks):
        gather(i).wait()
        scatter(i).start()
        nxt = i + _TC_NBUF - 1
        if nxt < nchunks:
            if nxt - _TC_NBUF >= 0:
                scatter(nxt - _TC_NBUF).wait()
            gather(nxt).start()
    for i in range(max(0, nchunks - _TC_NBUF), nchunks):
        scatter(i).wait()


def kernel(images_batch, masks_batch):
    del masks_batch
    B, ve_dim, feature_dim = images_batch.shape
    rows = B * ve_dim
    tc_rows = rows // 2
    flat = images_batch.reshape(rows, feature_dim)

    partial = _sc_part(flat, rows, feature_dim, tc_rows)

    out = pl.pallas_call(
        functools.partial(_tc_body, tc_rows),
        in_specs=[pl.BlockSpec(memory_space=pl.ANY),
                  pl.BlockSpec(memory_space=pl.ANY)],
        out_specs=pl.BlockSpec(memory_space=pl.ANY),
        scratch_shapes=(
            [pltpu.VMEM((_TC_CHUNK, feature_dim), jnp.float32)
             for _ in range(_TC_NBUF)]
            + [pltpu.SemaphoreType.DMA((_TC_NBUF,)),
               pltpu.SemaphoreType.DMA((_TC_NBUF,))]
        ),
        out_shape=jax.ShapeDtypeStruct((rows, feature_dim), flat.dtype),
        input_output_aliases={1: 0},
    )(flat, partial)
    return out.reshape(B, ve_dim, feature_dim)
